# baseline (speedup 1.0000x reference)
import jax
import jax.numpy as jnp
from jax.experimental import pallas as pl
from jax.experimental.pallas import tpu as pltpu

S = 2048
B = 4
CH = 128
NBUF = 4
NCH = S // CH


def _posemb_add_body(x_hbm, e_hbm, o_hbm, xbuf, ebuf, obuf, xsem, esem, osem):
    def in_x(c, slot):
        return pltpu.make_async_copy(
            x_hbm.at[pl.ds(c * CH, CH)], xbuf.at[slot], xsem.at[slot])

    def in_e(c, slot):
        return pltpu.make_async_copy(
            e_hbm.at[pl.ds((c * CH + 1) * 8, CH * 8)], ebuf.at[slot],
            esem.at[slot])

    def out_o(c, slot):
        return pltpu.make_async_copy(
            obuf.at[slot], o_hbm.at[pl.ds(c * CH, CH)], osem.at[slot])

    for s in range(NBUF):
        in_x(s, s).start()
        in_e(s, s).start()

    for c in range(NCH):
        slot = c % NBUF
        in_x(c, slot).wait()
        in_e(c, slot).wait()
        if c >= NBUF:
            out_o(c - NBUF, slot).wait()
        emb = ebuf[slot].reshape(CH, 8, 128)
        obuf[slot] = xbuf[slot] + emb[:, None, :, :]
        out_o(c, slot).start()
        if c + NBUF < NCH:
            in_x(c + NBUF, slot).start()
            in_e(c + NBUF, slot).start()

    for c in range(NCH - NBUF, NCH):
        out_o(c, c % NBUF).wait()


def kernel(inputs, pos_emb):
    S_, B_, D = inputs.shape
    T = pos_emb.shape[0]
    x4 = inputs.reshape(S_, B_, 8, D // 8)
    e2 = pos_emb.reshape(T * 8, D // 8)
    out = pl.pallas_call(
        _posemb_add_body,
        in_specs=[
            pl.BlockSpec(memory_space=pltpu.HBM),
            pl.BlockSpec(memory_space=pltpu.HBM),
        ],
        out_specs=pl.BlockSpec(memory_space=pltpu.HBM),
        out_shape=jax.ShapeDtypeStruct((S_, B_, 8, D // 8), inputs.dtype),
        scratch_shapes=[
            pltpu.VMEM((NBUF, CH, B, 8, 128), jnp.float32),
            pltpu.VMEM((NBUF, CH * 8, 128), jnp.float32),
            pltpu.VMEM((NBUF, CH, B, 8, 128), jnp.float32),
            pltpu.SemaphoreType.DMA((NBUF,)),
            pltpu.SemaphoreType.DMA((NBUF,)),
            pltpu.SemaphoreType.DMA((NBUF,)),
        ],
    )(x4, e2)
    return out.reshape(S_, B_, D)
